# Initial kernel scaffold; baseline (speedup 1.0000x reference)
import jax
import jax.numpy as jnp
from jax.experimental import pallas as pl


def kernel(x, pos, idx, dis, w1, g1, b1, w2, g2, b2, w1n, g1n, b1n, w2n, g2n, b2n):
    raise NotImplementedError("write your pallas kernel here")


# speedup vs baseline: 11.4189x; 11.4189x over previous
import functools

import jax
import jax.numpy as jnp
from jax import lax
from jax.experimental import pallas as pl
from jax.experimental.pallas import tpu as pltpu
from jax.experimental.pallas import tpu_sc as plsc

B, C_IN, N, K, CH = 8, 3, 16384, 20, 16
R = B * N
BNK = B * N * K
D = 8
LW = K * D
LE = K * CH
P = 512
GRID = R // P

NC, NS = 2, 16
NW = NC * NS
E_W = BNK // NW
SUP = 2048
INNER = 128
N_SUP = E_W // SUP


def _sc_gather(table, idx_flat):
    @functools.partial(
        pl.kernel,
        mesh=plsc.VectorSubcoreMesh(core_axis_name="c", subcore_axis_name="s"),
        out_type=jax.ShapeDtypeStruct((BNK, D), jnp.float32),
        scratch_types=[
            pltpu.VMEM((SUP,), jnp.int32),
            pltpu.VMEM((SUP, D), jnp.float32),
            pltpu.SemaphoreType.DMA,
        ],
        compiler_params=pltpu.CompilerParams(use_tc_tiling_on_sc=False),
    )
    def gather_kernel(table_hbm, idx_hbm, out_hbm, idx_v, rows_v, sem):
        wid = lax.axis_index("s") * NC + lax.axis_index("c")
        wbase = wid * E_W

        def body(i, carry):
            base = wbase + i * SUP
            pltpu.sync_copy(idx_hbm.at[pl.ds(base, SUP)], idx_v)
            copies = []
            for j in range(SUP // INNER):
                copies.append(
                    pltpu.async_copy(
                        table_hbm.at[idx_v.at[pl.ds(j * INNER, INNER)]],
                        rows_v.at[pl.ds(j * INNER, INNER)],
                        sem,
                    )
                )
            for c in copies:
                c.wait()
            pltpu.sync_copy(rows_v, out_hbm.at[pl.ds(base, SUP)])
            return carry

        lax.fori_loop(0, N_SUP, body, 0)

    return gather_kernel(table, idx_flat)


def _full(shape):
    return pl.BlockSpec(shape, lambda i: (0, 0))


def _rows(width):
    return pl.BlockSpec((P, width), lambda i: (i, 0))


def _pass_a_kernel(g_ref, x_ref, bd1_ref, bd1n_ref, wx_ref, wxn_ref, acc_ref):
    g = g_ref[...]
    x = x_ref[...]
    y1 = jnp.dot(g, bd1_ref[...], preferred_element_type=jnp.float32,
                 precision=lax.Precision.HIGHEST)
    y1 += jnp.dot(x, wx_ref[...], preferred_element_type=jnp.float32,
                  precision=lax.Precision.HIGHEST)
    y1n = jnp.dot(g, bd1n_ref[...], preferred_element_type=jnp.float32,
                  precision=lax.Precision.HIGHEST)
    y1n += jnp.dot(x, wxn_ref[...], preferred_element_type=jnp.float32,
                   precision=lax.Precision.HIGHEST)
    part = jnp.stack(
        [
            jnp.sum(y1, axis=0),
            jnp.sum(y1 * y1, axis=0),
            jnp.sum(y1n, axis=0),
            jnp.sum(y1n * y1n, axis=0),
        ]
    )
    part = jnp.concatenate([part, jnp.zeros((4, LE), jnp.float32)], axis=0)

    @pl.when(pl.program_id(0) == 0)
    def _():
        acc_ref[...] = jnp.zeros_like(acc_ref)

    acc_ref[...] += part


def _pass_b_kernel(g_ref, x_ref, bd1_ref, bd1n_ref, wx_ref, wxn_ref,
                   prm_ref, m2_ref, m2n_ref,
                   eo_ref, s2_ref, s2n_ref, acc_ref):
    g = g_ref[...]
    x = x_ref[...]
    y1 = jnp.dot(g, bd1_ref[...], preferred_element_type=jnp.float32,
                 precision=lax.Precision.HIGHEST)
    y1 += jnp.dot(x, wx_ref[...], preferred_element_type=jnp.float32,
                  precision=lax.Precision.HIGHEST)
    y1n = jnp.dot(g, bd1n_ref[...], preferred_element_type=jnp.float32,
                  precision=lax.Precision.HIGHEST)
    y1n += jnp.dot(x, wxn_ref[...], preferred_element_type=jnp.float32,
                   precision=lax.Precision.HIGHEST)
    xa = y1 * prm_ref[0:1, :] + prm_ref[1:2, :]
    xa = jnp.where(xa > 0, xa, 0.2 * xa)
    eo = y1n * prm_ref[2:3, :] + prm_ref[3:4, :]
    eo = jnp.where(eo > 0, eo, 0.2 * eo)
    eo_ref[...] = eo
    s2 = jnp.dot(xa, m2_ref[...], preferred_element_type=jnp.float32,
                 precision=lax.Precision.HIGHEST)
    s2n = jnp.dot(eo, m2n_ref[...], preferred_element_type=jnp.float32,
                  precision=lax.Precision.HIGHEST)
    s2_ref[...] = s2
    s2n_ref[...] = s2n
    sums = jnp.stack(
        [jnp.sum(s2), jnp.sum(s2 * s2), jnp.sum(s2n), jnp.sum(s2n * s2n)]
    )
    part = jnp.concatenate(
        [jnp.broadcast_to(sums[:, None], (4, 128)),
         jnp.zeros((4, 128), jnp.float32)], axis=0)

    @pl.when(pl.program_id(0) == 0)
    def _():
        acc_ref[...] = jnp.zeros_like(acc_ref)

    acc_ref[...] += part


def _pass_c_kernel(g_ref, x_ref, bd1n_ref, wxn_ref, prm_ref, scal_ref,
                   s2_ref, s2n_ref, exp_ref, msum_ref, out_ref):
    g = g_ref[...]
    x = x_ref[...]
    y1n = jnp.dot(g, bd1n_ref[...], preferred_element_type=jnp.float32,
                  precision=lax.Precision.HIGHEST)
    y1n += jnp.dot(x, wxn_ref[...], preferred_element_type=jnp.float32,
                   precision=lax.Precision.HIGHEST)
    eo = y1n * prm_ref[2:3, :] + prm_ref[3:4, :]
    eo = jnp.where(eo > 0, eo, 0.2 * eo)
    x2 = s2_ref[...] * scal_ref[0:1, 0:1] + scal_ref[0:1, 1:2]
    e2 = s2n_ref[...] * scal_ref[0:1, 2:3] + scal_ref[0:1, 3:4]
    att = x2 + e2
    att = jnp.where(att > 0, att, 0.2 * att)
    att = att - jnp.max(att, axis=1, keepdims=True)
    att = jnp.exp(att)
    att = att / jnp.sum(att, axis=1, keepdims=True)
    att_x = jnp.dot(att, exp_ref[...], preferred_element_type=jnp.float32,
                    precision=lax.Precision.HIGHEST)
    out_ref[...] = jnp.dot(att_x * eo, msum_ref[...],
                           preferred_element_type=jnp.float32,
                           precision=lax.Precision.HIGHEST)


def kernel(x, pos, idx, dis, w1, g1, b1, w2, g2, b2, w1n, g1n, b1n, w2n, g2n, b2n):
    f32 = jnp.float32
    xt = jnp.transpose(x, (0, 2, 1)).reshape(R, C_IN)
    table = jnp.pad(xt, ((0, 0), (0, D - C_IN)))
    idx_flat = (idx + (jnp.arange(B, dtype=idx.dtype) * N)[:, None, None]).reshape(-1)

    w1a_t = jnp.pad(w1[:, :C_IN].T, ((0, D - C_IN), (0, 0)))
    w1na_t = jnp.pad(w1n[:, :C_IN].T, ((0, D - C_IN), (0, 0)))
    d1_t = jnp.pad((w1[:, C_IN:] - w1[:, :C_IN]).T, ((0, D - C_IN), (0, 0)))
    d1n_t = jnp.pad((w1n[:, C_IN:] - w1n[:, :C_IN]).T, ((0, D - C_IN), (0, 0)))
    eye_k = jnp.eye(K, dtype=f32)
    bd1 = jnp.kron(eye_k, w1a_t)
    bd1n = jnp.kron(eye_k, w1na_t)
    wx = jnp.tile(d1_t, (1, K))
    wxn = jnp.tile(d1n_t, (1, K))
    m2 = jnp.kron(eye_k, w2[0][:, None])
    m2n = jnp.kron(eye_k, w2n[0][:, None])
    expand = jnp.kron(eye_k, jnp.ones((1, CH), f32))
    msum = jnp.kron(jnp.ones((K, 1), f32), jnp.eye(CH, dtype=f32))

    g_flat = _sc_gather(table, idx_flat)
    g2d = g_flat.reshape(R, LW)

    acc1 = pl.pallas_call(
        _pass_a_kernel,
        grid=(GRID,),
        in_specs=[_rows(LW), _rows(D), _full((LW, LE)), _full((LW, LE)),
                  _full((D, LE)), _full((D, LE))],
        out_specs=_full((8, LE)),
        out_shape=jax.ShapeDtypeStruct((8, LE), f32),
    )(g2d, table, bd1, bd1n, wx, wxn)

    m = f32(BNK)
    sum1 = acc1[0].reshape(K, CH).sum(0)
    sq1 = acc1[1].reshape(K, CH).sum(0)
    sum1n = acc1[2].reshape(K, CH).sum(0)
    sq1n = acc1[3].reshape(K, CH).sum(0)
    mean1 = sum1 / m
    var1 = sq1 / m - mean1 * mean1
    mean1n = sum1n / m
    var1n = sq1n / m - mean1n * mean1n
    sc1 = g1 / jnp.sqrt(var1 + 1e-5)
    sh1 = b1 - mean1 * sc1
    sc1n = g1n / jnp.sqrt(var1n + 1e-5)
    sh1n = b1n - mean1n * sc1n
    prm = jnp.stack([jnp.tile(sc1, K), jnp.tile(sh1, K),
                     jnp.tile(sc1n, K), jnp.tile(sh1n, K)])

    eo2d, s2, s2n, acc2 = pl.pallas_call(
        _pass_b_kernel,
        grid=(GRID,),
        in_specs=[_rows(LW), _rows(D), _full((LW, LE)), _full((LW, LE)),
                  _full((D, LE)), _full((D, LE)), _full((4, LE)),
                  _full((LE, K)), _full((LE, K))],
        out_specs=[_rows(LE), _rows(K), _rows(K), _full((8, 128))],
        out_shape=[
            jax.ShapeDtypeStruct((R, LE), f32),
            jax.ShapeDtypeStruct((R, K), f32),
            jax.ShapeDtypeStruct((R, K), f32),
            jax.ShapeDtypeStruct((8, 128), f32),
        ],
    )(g2d, table, bd1, bd1n, wx, wxn, prm, m2, m2n)

    mean2 = acc2[0, 0] / m
    var2 = acc2[1, 0] / m - mean2 * mean2
    mean2n = acc2[2, 0] / m
    var2n = acc2[3, 0] / m - mean2n * mean2n
    a2 = g2[0] / jnp.sqrt(var2 + 1e-5)
    c2 = b2[0] - mean2 * a2
    a2n = g2n[0] / jnp.sqrt(var2n + 1e-5)
    c2n = b2n[0] - mean2n * a2n
    scal = jnp.zeros((1, 128), f32)
    scal = scal.at[0, 0].set(a2).at[0, 1].set(c2)
    scal = scal.at[0, 2].set(a2n).at[0, 3].set(c2n)

    out2d = pl.pallas_call(
        _pass_c_kernel,
        grid=(GRID,),
        in_specs=[_rows(LW), _rows(D), _full((LW, LE)), _full((D, LE)),
                  _full((4, LE)), _full((1, 128)), _rows(K), _rows(K),
                  _full((K, LE)), _full((LE, CH))],
        out_specs=_rows(CH),
        out_shape=jax.ShapeDtypeStruct((R, CH), f32),
    )(g2d, table, bd1n, wxn, prm, scal, s2, s2n, expand, msum)

    out = out2d.reshape(B, N, CH)
    edge_o = eo2d.reshape(B, N, K, CH)
    return (out, edge_o)


# speedup vs baseline: 11.5642x; 1.0127x over previous
import functools

import jax
import jax.numpy as jnp
from jax import lax
from jax.experimental import pallas as pl
from jax.experimental.pallas import tpu as pltpu
from jax.experimental.pallas import tpu_sc as plsc

B, C_IN, N, K, CH = 8, 3, 16384, 20, 16
R = B * N
BNK = B * N * K
D = 8
LW = K * D
LE = K * CH
P = 512
GRID = R // P

NC, NS = 2, 16
NW = NC * NS
E_W = BNK // NW
SUP = 2048
INNER = 128
N_SUP = E_W // SUP


def _sc_gather(table, idx_flat):
    @functools.partial(
        pl.kernel,
        mesh=plsc.VectorSubcoreMesh(core_axis_name="c", subcore_axis_name="s"),
        out_type=jax.ShapeDtypeStruct((BNK, D), jnp.float32),
        scratch_types=[
            pltpu.VMEM((SUP,), jnp.int32),
            pltpu.VMEM((SUP,), jnp.int32),
            pltpu.VMEM((SUP, D), jnp.float32),
            pltpu.VMEM((SUP, D), jnp.float32),
            pltpu.SemaphoreType.DMA,
            pltpu.SemaphoreType.DMA,
            pltpu.SemaphoreType.DMA,
        ],
        compiler_params=pltpu.CompilerParams(use_tc_tiling_on_sc=False),
    )
    def gather_kernel(table_hbm, idx_hbm, out_hbm,
                      ib0, ib1, rb0, rb1, si, sg, ss):
        wid = lax.axis_index("s") * NC + lax.axis_index("c")
        wbase = wid * E_W
        ibufs = (ib0, ib1)
        rbufs = (rb0, rb1)

        def start_idx(c, buf):
            pltpu.async_copy(idx_hbm.at[pl.ds(wbase + c * SUP, SUP)],
                             ibufs[buf], si)

        def wait_idx(buf):
            pltpu.make_async_copy(idx_hbm.at[pl.ds(0, SUP)],
                                  ibufs[buf], si).wait()

        def start_store(c, buf):
            pltpu.async_copy(rbufs[buf],
                             out_hbm.at[pl.ds(wbase + c * SUP, SUP)], ss)

        def wait_store(buf):
            pltpu.make_async_copy(rbufs[buf],
                                  out_hbm.at[pl.ds(0, SUP)], ss).wait()

        def run_gathers(buf):
            copies = []
            for j in range(SUP // INNER):
                copies.append(
                    pltpu.async_copy(
                        table_hbm.at[ibufs[buf].at[pl.ds(j * INNER, INNER)]],
                        rbufs[buf].at[pl.ds(j * INNER, INNER)],
                        sg,
                    )
                )
            for c in copies:
                c.wait()

        def process(c, buf, prefetch, storewait):
            wait_idx(buf)
            if prefetch:
                start_idx(c + 1, 1 - buf)
            if storewait:
                wait_store(buf)
            run_gathers(buf)
            start_store(c, buf)

        start_idx(0, 0)
        process(0, 0, True, False)
        process(1, 1, True, False)

        def body(j, carry):
            process(2 * j, 0, True, True)
            process(2 * j + 1, 1, True, True)
            return carry

        lax.fori_loop(1, N_SUP // 2 - 1, body, 0)

        process(N_SUP - 2, 0, True, True)
        process(N_SUP - 1, 1, False, True)
        wait_store(0)
        wait_store(1)

    return gather_kernel(table, idx_flat)


def _full(shape):
    return pl.BlockSpec(shape, lambda i: (0, 0))


def _rows(width):
    return pl.BlockSpec((P, width), lambda i: (i, 0))


def _pass_a_kernel(g_ref, x_ref, bd1_ref, bd1n_ref, wx_ref, wxn_ref, acc_ref):
    g = g_ref[...]
    x = x_ref[...]
    y1 = jnp.dot(g, bd1_ref[...], preferred_element_type=jnp.float32,
                 precision=lax.Precision.HIGHEST)
    y1 += jnp.dot(x, wx_ref[...], preferred_element_type=jnp.float32,
                  precision=lax.Precision.HIGHEST)
    y1n = jnp.dot(g, bd1n_ref[...], preferred_element_type=jnp.float32,
                  precision=lax.Precision.HIGHEST)
    y1n += jnp.dot(x, wxn_ref[...], preferred_element_type=jnp.float32,
                   precision=lax.Precision.HIGHEST)
    part = jnp.stack(
        [
            jnp.sum(y1, axis=0),
            jnp.sum(y1 * y1, axis=0),
            jnp.sum(y1n, axis=0),
            jnp.sum(y1n * y1n, axis=0),
        ]
    )
    part = jnp.concatenate([part, jnp.zeros((4, LE), jnp.float32)], axis=0)

    @pl.when(pl.program_id(0) == 0)
    def _():
        acc_ref[...] = jnp.zeros_like(acc_ref)

    acc_ref[...] += part


def _pass_b_kernel(g_ref, x_ref, bd1_ref, bd1n_ref, wx_ref, wxn_ref,
                   prm_ref, m2_ref, m2n_ref,
                   eo_ref, s2_ref, s2n_ref, acc_ref):
    g = g_ref[...]
    x = x_ref[...]
    y1 = jnp.dot(g, bd1_ref[...], preferred_element_type=jnp.float32,
                 precision=lax.Precision.HIGHEST)
    y1 += jnp.dot(x, wx_ref[...], preferred_element_type=jnp.float32,
                  precision=lax.Precision.HIGHEST)
    y1n = jnp.dot(g, bd1n_ref[...], preferred_element_type=jnp.float32,
                  precision=lax.Precision.HIGHEST)
    y1n += jnp.dot(x, wxn_ref[...], preferred_element_type=jnp.float32,
                   precision=lax.Precision.HIGHEST)
    xa = y1 * prm_ref[0:1, :] + prm_ref[1:2, :]
    xa = jnp.where(xa > 0, xa, 0.2 * xa)
    eo = y1n * prm_ref[2:3, :] + prm_ref[3:4, :]
    eo = jnp.where(eo > 0, eo, 0.2 * eo)
    eo_ref[...] = eo
    s2 = jnp.dot(xa, m2_ref[...], preferred_element_type=jnp.float32,
                 precision=lax.Precision.HIGHEST)
    s2n = jnp.dot(eo, m2n_ref[...], preferred_element_type=jnp.float32,
                  precision=lax.Precision.HIGHEST)
    s2_ref[...] = s2
    s2n_ref[...] = s2n
    sums = jnp.stack(
        [jnp.sum(s2), jnp.sum(s2 * s2), jnp.sum(s2n), jnp.sum(s2n * s2n)]
    )
    part = jnp.concatenate(
        [jnp.broadcast_to(sums[:, None], (4, 128)),
         jnp.zeros((4, 128), jnp.float32)], axis=0)

    @pl.when(pl.program_id(0) == 0)
    def _():
        acc_ref[...] = jnp.zeros_like(acc_ref)

    acc_ref[...] += part


def _pass_c_kernel(g_ref, x_ref, bd1n_ref, wxn_ref, prm_ref, scal_ref,
                   s2_ref, s2n_ref, exp_ref, msum_ref, out_ref):
    g = g_ref[...]
    x = x_ref[...]
    y1n = jnp.dot(g, bd1n_ref[...], preferred_element_type=jnp.float32,
                  precision=lax.Precision.HIGHEST)
    y1n += jnp.dot(x, wxn_ref[...], preferred_element_type=jnp.float32,
                   precision=lax.Precision.HIGHEST)
    eo = y1n * prm_ref[2:3, :] + prm_ref[3:4, :]
    eo = jnp.where(eo > 0, eo, 0.2 * eo)
    x2 = s2_ref[...] * scal_ref[0:1, 0:1] + scal_ref[0:1, 1:2]
    e2 = s2n_ref[...] * scal_ref[0:1, 2:3] + scal_ref[0:1, 3:4]
    att = x2 + e2
    att = jnp.where(att > 0, att, 0.2 * att)
    att = att - jnp.max(att, axis=1, keepdims=True)
    att = jnp.exp(att)
    att = att / jnp.sum(att, axis=1, keepdims=True)
    att_x = jnp.dot(att, exp_ref[...], preferred_element_type=jnp.float32,
                    precision=lax.Precision.HIGHEST)
    out_ref[...] = jnp.dot(att_x * eo, msum_ref[...],
                           preferred_element_type=jnp.float32,
                           precision=lax.Precision.HIGHEST)


def kernel(x, pos, idx, dis, w1, g1, b1, w2, g2, b2, w1n, g1n, b1n, w2n, g2n, b2n):
    f32 = jnp.float32
    xt = jnp.transpose(x, (0, 2, 1)).reshape(R, C_IN)
    table = jnp.pad(xt, ((0, 0), (0, D - C_IN)))
    idx_flat = (idx + (jnp.arange(B, dtype=idx.dtype) * N)[:, None, None]).reshape(-1)

    w1a_t = jnp.pad(w1[:, :C_IN].T, ((0, D - C_IN), (0, 0)))
    w1na_t = jnp.pad(w1n[:, :C_IN].T, ((0, D - C_IN), (0, 0)))
    d1_t = jnp.pad((w1[:, C_IN:] - w1[:, :C_IN]).T, ((0, D - C_IN), (0, 0)))
    d1n_t = jnp.pad((w1n[:, C_IN:] - w1n[:, :C_IN]).T, ((0, D - C_IN), (0, 0)))
    eye_k = jnp.eye(K, dtype=f32)
    bd1 = jnp.kron(eye_k, w1a_t)
    bd1n = jnp.kron(eye_k, w1na_t)
    wx = jnp.tile(d1_t, (1, K))
    wxn = jnp.tile(d1n_t, (1, K))
    m2 = jnp.kron(eye_k, w2[0][:, None])
    m2n = jnp.kron(eye_k, w2n[0][:, None])
    expand = jnp.kron(eye_k, jnp.ones((1, CH), f32))
    msum = jnp.kron(jnp.ones((K, 1), f32), jnp.eye(CH, dtype=f32))

    g_flat = _sc_gather(table, idx_flat)
    g2d = g_flat.reshape(R, LW)

    acc1 = pl.pallas_call(
        _pass_a_kernel,
        grid=(GRID,),
        in_specs=[_rows(LW), _rows(D), _full((LW, LE)), _full((LW, LE)),
                  _full((D, LE)), _full((D, LE))],
        out_specs=_full((8, LE)),
        out_shape=jax.ShapeDtypeStruct((8, LE), f32),
    )(g2d, table, bd1, bd1n, wx, wxn)

    m = f32(BNK)
    sum1 = acc1[0].reshape(K, CH).sum(0)
    sq1 = acc1[1].reshape(K, CH).sum(0)
    sum1n = acc1[2].reshape(K, CH).sum(0)
    sq1n = acc1[3].reshape(K, CH).sum(0)
    mean1 = sum1 / m
    var1 = sq1 / m - mean1 * mean1
    mean1n = sum1n / m
    var1n = sq1n / m - mean1n * mean1n
    sc1 = g1 / jnp.sqrt(var1 + 1e-5)
    sh1 = b1 - mean1 * sc1
    sc1n = g1n / jnp.sqrt(var1n + 1e-5)
    sh1n = b1n - mean1n * sc1n
    prm = jnp.stack([jnp.tile(sc1, K), jnp.tile(sh1, K),
                     jnp.tile(sc1n, K), jnp.tile(sh1n, K)])

    eo2d, s2, s2n, acc2 = pl.pallas_call(
        _pass_b_kernel,
        grid=(GRID,),
        in_specs=[_rows(LW), _rows(D), _full((LW, LE)), _full((LW, LE)),
                  _full((D, LE)), _full((D, LE)), _full((4, LE)),
                  _full((LE, K)), _full((LE, K))],
        out_specs=[_rows(LE), _rows(K), _rows(K), _full((8, 128))],
        out_shape=[
            jax.ShapeDtypeStruct((R, LE), f32),
            jax.ShapeDtypeStruct((R, K), f32),
            jax.ShapeDtypeStruct((R, K), f32),
            jax.ShapeDtypeStruct((8, 128), f32),
        ],
    )(g2d, table, bd1, bd1n, wx, wxn, prm, m2, m2n)

    mean2 = acc2[0, 0] / m
    var2 = acc2[1, 0] / m - mean2 * mean2
    mean2n = acc2[2, 0] / m
    var2n = acc2[3, 0] / m - mean2n * mean2n
    a2 = g2[0] / jnp.sqrt(var2 + 1e-5)
    c2 = b2[0] - mean2 * a2
    a2n = g2n[0] / jnp.sqrt(var2n + 1e-5)
    c2n = b2n[0] - mean2n * a2n
    scal = jnp.zeros((1, 128), f32)
    scal = scal.at[0, 0].set(a2).at[0, 1].set(c2)
    scal = scal.at[0, 2].set(a2n).at[0, 3].set(c2n)

    out2d = pl.pallas_call(
        _pass_c_kernel,
        grid=(GRID,),
        in_specs=[_rows(LW), _rows(D), _full((LW, LE)), _full((D, LE)),
                  _full((4, LE)), _full((1, 128)), _rows(K), _rows(K),
                  _full((K, LE)), _full((LE, CH))],
        out_specs=_rows(CH),
        out_shape=jax.ShapeDtypeStruct((R, CH), f32),
    )(g2d, table, bd1n, wxn, prm, scal, s2, s2n, expand, msum)

    out = out2d.reshape(B, N, CH)
    edge_o = eo2d.reshape(B, N, K, CH)
    return (out, edge_o)


# speedup vs baseline: 24.0653x; 2.0810x over previous
import functools

import jax
import jax.numpy as jnp
from jax import lax
from jax.experimental import pallas as pl
from jax.experimental.pallas import tpu as pltpu
from jax.experimental.pallas import tpu_sc as plsc

B, C_IN, N, K, CH = 8, 3, 16384, 20, 16
R = B * N
BNK = B * N * K
D = 8
LW = K * D
LE = K * CH
P = 512
GRID = R // P

NC, NS = 2, 16
NW = NC * NS
E_W = BNK // NW
SUP = 2048
INNER = 128
N_SUP = E_W // SUP


def _sc_gather(table, idx_flat):
    @functools.partial(
        pl.kernel,
        mesh=plsc.VectorSubcoreMesh(core_axis_name="c", subcore_axis_name="s"),
        out_type=jax.ShapeDtypeStruct((BNK, D), jnp.float32),
        scratch_types=[
            pltpu.VMEM((SUP,), jnp.int32),
            pltpu.VMEM((SUP,), jnp.int32),
            pltpu.VMEM((SUP, D), jnp.float32),
            pltpu.VMEM((SUP, D), jnp.float32),
            pltpu.SemaphoreType.DMA,
            pltpu.SemaphoreType.DMA,
            pltpu.SemaphoreType.DMA,
        ],
        compiler_params=pltpu.CompilerParams(use_tc_tiling_on_sc=False),
    )
    def gather_kernel(table_hbm, idx_hbm, out_hbm,
                      ib0, ib1, rb0, rb1, si, sg, ss):
        wid = lax.axis_index("s") * NC + lax.axis_index("c")
        wbase = wid * E_W
        ibufs = (ib0, ib1)
        rbufs = (rb0, rb1)

        def start_idx(c, buf):
            pltpu.async_copy(idx_hbm.at[pl.ds(wbase + c * SUP, SUP)],
                             ibufs[buf], si)

        def wait_idx(buf):
            pltpu.make_async_copy(idx_hbm.at[pl.ds(0, SUP)],
                                  ibufs[buf], si).wait()

        def start_store(c, buf):
            pltpu.async_copy(rbufs[buf],
                             out_hbm.at[pl.ds(wbase + c * SUP, SUP)], ss)

        def wait_store(buf):
            pltpu.make_async_copy(rbufs[buf],
                                  out_hbm.at[pl.ds(0, SUP)], ss).wait()

        def run_gathers(buf):
            copies = []
            for j in range(SUP // INNER):
                copies.append(
                    pltpu.async_copy(
                        table_hbm.at[ibufs[buf].at[pl.ds(j * INNER, INNER)]],
                        rbufs[buf].at[pl.ds(j * INNER, INNER)],
                        sg,
                    )
                )
            for c in copies:
                c.wait()

        def process(c, buf, prefetch, storewait):
            wait_idx(buf)
            if prefetch:
                start_idx(c + 1, 1 - buf)
            if storewait:
                wait_store(buf)
            run_gathers(buf)
            start_store(c, buf)

        start_idx(0, 0)
        process(0, 0, True, False)
        process(1, 1, True, False)

        def body(j, carry):
            process(2 * j, 0, True, True)
            process(2 * j + 1, 1, True, True)
            return carry

        lax.fori_loop(1, N_SUP // 2 - 1, body, 0)

        process(N_SUP - 2, 0, True, True)
        process(N_SUP - 1, 1, False, True)
        wait_store(0)
        wait_store(1)

    return gather_kernel(table, idx_flat)


def _full(shape):
    return pl.BlockSpec(shape, lambda i: (0, 0))


def _rows(width):
    return pl.BlockSpec((P, width), lambda i: (i, 0))


def _pass_a_kernel(g_ref, x_ref, bd1_ref, bd1n_ref, wx_ref, wxn_ref, acc_ref):
    g = g_ref[...]
    x = x_ref[...]
    y1 = jnp.dot(g, bd1_ref[...], preferred_element_type=jnp.float32,
                 precision=lax.Precision.DEFAULT)
    y1 += jnp.dot(x, wx_ref[...], preferred_element_type=jnp.float32,
                  precision=lax.Precision.DEFAULT)
    y1n = jnp.dot(g, bd1n_ref[...], preferred_element_type=jnp.float32,
                  precision=lax.Precision.DEFAULT)
    y1n += jnp.dot(x, wxn_ref[...], preferred_element_type=jnp.float32,
                   precision=lax.Precision.DEFAULT)
    part = jnp.stack(
        [
            jnp.sum(y1, axis=0),
            jnp.sum(y1 * y1, axis=0),
            jnp.sum(y1n, axis=0),
            jnp.sum(y1n * y1n, axis=0),
        ]
    )
    part = jnp.concatenate([part, jnp.zeros((4, LE), jnp.float32)], axis=0)

    @pl.when(pl.program_id(0) == 0)
    def _():
        acc_ref[...] = jnp.zeros_like(acc_ref)

    acc_ref[...] += part


def _pass_b_kernel(g_ref, x_ref, bd1_ref, bd1n_ref, wx_ref, wxn_ref,
                   prm_ref, m2_ref, m2n_ref,
                   eo_ref, s2_ref, s2n_ref, acc_ref):
    g = g_ref[...]
    x = x_ref[...]
    y1 = jnp.dot(g, bd1_ref[...], preferred_element_type=jnp.float32,
                 precision=lax.Precision.DEFAULT)
    y1 += jnp.dot(x, wx_ref[...], preferred_element_type=jnp.float32,
                  precision=lax.Precision.DEFAULT)
    y1n = jnp.dot(g, bd1n_ref[...], preferred_element_type=jnp.float32,
                  precision=lax.Precision.DEFAULT)
    y1n += jnp.dot(x, wxn_ref[...], preferred_element_type=jnp.float32,
                   precision=lax.Precision.DEFAULT)
    xa = y1 * prm_ref[0:1, :] + prm_ref[1:2, :]
    xa = jnp.where(xa > 0, xa, 0.2 * xa)
    eo = y1n * prm_ref[2:3, :] + prm_ref[3:4, :]
    eo = jnp.where(eo > 0, eo, 0.2 * eo)
    eo_ref[...] = eo
    s2 = jnp.dot(xa, m2_ref[...], preferred_element_type=jnp.float32,
                 precision=lax.Precision.DEFAULT)
    s2n = jnp.dot(eo, m2n_ref[...], preferred_element_type=jnp.float32,
                  precision=lax.Precision.DEFAULT)
    s2_ref[...] = s2
    s2n_ref[...] = s2n
    sums = jnp.stack(
        [jnp.sum(s2), jnp.sum(s2 * s2), jnp.sum(s2n), jnp.sum(s2n * s2n)]
    )
    part = jnp.concatenate(
        [jnp.broadcast_to(sums[:, None], (4, 128)),
         jnp.zeros((4, 128), jnp.float32)], axis=0)

    @pl.when(pl.program_id(0) == 0)
    def _():
        acc_ref[...] = jnp.zeros_like(acc_ref)

    acc_ref[...] += part


def _pass_c_kernel(g_ref, x_ref, bd1n_ref, wxn_ref, prm_ref, scal_ref,
                   s2_ref, s2n_ref, exp_ref, msum_ref, out_ref):
    g = g_ref[...]
    x = x_ref[...]
    y1n = jnp.dot(g, bd1n_ref[...], preferred_element_type=jnp.float32,
                  precision=lax.Precision.DEFAULT)
    y1n += jnp.dot(x, wxn_ref[...], preferred_element_type=jnp.float32,
                   precision=lax.Precision.DEFAULT)
    eo = y1n * prm_ref[2:3, :] + prm_ref[3:4, :]
    eo = jnp.where(eo > 0, eo, 0.2 * eo)
    x2 = s2_ref[...] * scal_ref[0:1, 0:1] + scal_ref[0:1, 1:2]
    e2 = s2n_ref[...] * scal_ref[0:1, 2:3] + scal_ref[0:1, 3:4]
    att = x2 + e2
    att = jnp.where(att > 0, att, 0.2 * att)
    att = att - jnp.max(att, axis=1, keepdims=True)
    att = jnp.exp(att)
    att = att / jnp.sum(att, axis=1, keepdims=True)
    att_x = jnp.dot(att, exp_ref[...], preferred_element_type=jnp.float32,
                    precision=lax.Precision.DEFAULT)
    out_ref[...] = jnp.dot(att_x * eo, msum_ref[...],
                           preferred_element_type=jnp.float32,
                           precision=lax.Precision.DEFAULT)


def kernel(x, pos, idx, dis, w1, g1, b1, w2, g2, b2, w1n, g1n, b1n, w2n, g2n, b2n):
    f32 = jnp.float32
    xt = jnp.transpose(x, (0, 2, 1)).reshape(R, C_IN)
    table = jnp.pad(xt, ((0, 0), (0, D - C_IN)))
    idx_flat = (idx + (jnp.arange(B, dtype=idx.dtype) * N)[:, None, None]).reshape(-1)

    w1a_t = jnp.pad(w1[:, :C_IN].T, ((0, D - C_IN), (0, 0)))
    w1na_t = jnp.pad(w1n[:, :C_IN].T, ((0, D - C_IN), (0, 0)))
    d1_t = jnp.pad((w1[:, C_IN:] - w1[:, :C_IN]).T, ((0, D - C_IN), (0, 0)))
    d1n_t = jnp.pad((w1n[:, C_IN:] - w1n[:, :C_IN]).T, ((0, D - C_IN), (0, 0)))
    eye_k = jnp.eye(K, dtype=f32)
    bd1 = jnp.kron(eye_k, w1a_t)
    bd1n = jnp.kron(eye_k, w1na_t)
    wx = jnp.tile(d1_t, (1, K))
    wxn = jnp.tile(d1n_t, (1, K))
    m2 = jnp.kron(eye_k, w2[0][:, None])
    m2n = jnp.kron(eye_k, w2n[0][:, None])
    expand = jnp.kron(eye_k, jnp.ones((1, CH), f32))
    msum = jnp.kron(jnp.ones((K, 1), f32), jnp.eye(CH, dtype=f32))

    g_flat = _sc_gather(table, idx_flat)
    g2d = g_flat.reshape(R, LW)

    acc1 = pl.pallas_call(
        _pass_a_kernel,
        grid=(GRID,),
        in_specs=[_rows(LW), _rows(D), _full((LW, LE)), _full((LW, LE)),
                  _full((D, LE)), _full((D, LE))],
        out_specs=_full((8, LE)),
        out_shape=jax.ShapeDtypeStruct((8, LE), f32),
    )(g2d, table, bd1, bd1n, wx, wxn)

    m = f32(BNK)
    sum1 = acc1[0].reshape(K, CH).sum(0)
    sq1 = acc1[1].reshape(K, CH).sum(0)
    sum1n = acc1[2].reshape(K, CH).sum(0)
    sq1n = acc1[3].reshape(K, CH).sum(0)
    mean1 = sum1 / m
    var1 = sq1 / m - mean1 * mean1
    mean1n = sum1n / m
    var1n = sq1n / m - mean1n * mean1n
    sc1 = g1 / jnp.sqrt(var1 + 1e-5)
    sh1 = b1 - mean1 * sc1
    sc1n = g1n / jnp.sqrt(var1n + 1e-5)
    sh1n = b1n - mean1n * sc1n
    prm = jnp.stack([jnp.tile(sc1, K), jnp.tile(sh1, K),
                     jnp.tile(sc1n, K), jnp.tile(sh1n, K)])

    eo2d, s2, s2n, acc2 = pl.pallas_call(
        _pass_b_kernel,
        grid=(GRID,),
        in_specs=[_rows(LW), _rows(D), _full((LW, LE)), _full((LW, LE)),
                  _full((D, LE)), _full((D, LE)), _full((4, LE)),
                  _full((LE, K)), _full((LE, K))],
        out_specs=[_rows(LE), _rows(K), _rows(K), _full((8, 128))],
        out_shape=[
            jax.ShapeDtypeStruct((R, LE), f32),
            jax.ShapeDtypeStruct((R, K), f32),
            jax.ShapeDtypeStruct((R, K), f32),
            jax.ShapeDtypeStruct((8, 128), f32),
        ],
    )(g2d, table, bd1, bd1n, wx, wxn, prm, m2, m2n)

    mean2 = acc2[0, 0] / m
    var2 = acc2[1, 0] / m - mean2 * mean2
    mean2n = acc2[2, 0] / m
    var2n = acc2[3, 0] / m - mean2n * mean2n
    a2 = g2[0] / jnp.sqrt(var2 + 1e-5)
    c2 = b2[0] - mean2 * a2
    a2n = g2n[0] / jnp.sqrt(var2n + 1e-5)
    c2n = b2n[0] - mean2n * a2n
    scal = jnp.zeros((1, 128), f32)
    scal = scal.at[0, 0].set(a2).at[0, 1].set(c2)
    scal = scal.at[0, 2].set(a2n).at[0, 3].set(c2n)

    out2d = pl.pallas_call(
        _pass_c_kernel,
        grid=(GRID,),
        in_specs=[_rows(LW), _rows(D), _full((LW, LE)), _full((D, LE)),
                  _full((4, LE)), _full((1, 128)), _rows(K), _rows(K),
                  _full((K, LE)), _full((LE, CH))],
        out_specs=_rows(CH),
        out_shape=jax.ShapeDtypeStruct((R, CH), f32),
    )(g2d, table, bd1n, wxn, prm, scal, s2, s2n, expand, msum)

    out = out2d.reshape(B, N, CH)
    edge_o = eo2d.reshape(B, N, K, CH)
    return (out, edge_o)


# speedup vs baseline: 27.7539x; 1.1533x over previous
import functools

import jax
import jax.numpy as jnp
from jax import lax
from jax.experimental import pallas as pl
from jax.experimental.pallas import tpu as pltpu
from jax.experimental.pallas import tpu_sc as plsc

B, C_IN, N, K, CH = 8, 3, 16384, 20, 16
R = B * N
BNK = B * N * K
D = 8
LW = K * D
LE = K * CH
P = 1024
GRID = R // P

NC, NS = 2, 16
NW = NC * NS
E_W = BNK // NW
SUP = 2048
INNER = 128
N_SUP = E_W // SUP


def _sc_gather(table, idx_flat):
    @functools.partial(
        pl.kernel,
        mesh=plsc.VectorSubcoreMesh(core_axis_name="c", subcore_axis_name="s"),
        out_type=jax.ShapeDtypeStruct((BNK, D), jnp.float32),
        scratch_types=[
            pltpu.VMEM((SUP,), jnp.int32),
            pltpu.VMEM((SUP,), jnp.int32),
            pltpu.VMEM((SUP, D), jnp.float32),
            pltpu.VMEM((SUP, D), jnp.float32),
            pltpu.SemaphoreType.DMA,
            pltpu.SemaphoreType.DMA,
            pltpu.SemaphoreType.DMA,
        ],
        compiler_params=pltpu.CompilerParams(use_tc_tiling_on_sc=False),
    )
    def gather_kernel(table_hbm, idx_hbm, out_hbm,
                      ib0, ib1, rb0, rb1, si, sg, ss):
        wid = lax.axis_index("s") * NC + lax.axis_index("c")
        wbase = wid * E_W
        ibufs = (ib0, ib1)
        rbufs = (rb0, rb1)

        def start_idx(c, buf):
            pltpu.async_copy(idx_hbm.at[pl.ds(wbase + c * SUP, SUP)],
                             ibufs[buf], si)

        def wait_idx(buf):
            pltpu.make_async_copy(idx_hbm.at[pl.ds(0, SUP)],
                                  ibufs[buf], si).wait()

        def start_store(c, buf):
            pltpu.async_copy(rbufs[buf],
                             out_hbm.at[pl.ds(wbase + c * SUP, SUP)], ss)

        def wait_store(buf):
            pltpu.make_async_copy(rbufs[buf],
                                  out_hbm.at[pl.ds(0, SUP)], ss).wait()

        def run_gathers(buf):
            copies = []
            for j in range(SUP // INNER):
                copies.append(
                    pltpu.async_copy(
                        table_hbm.at[ibufs[buf].at[pl.ds(j * INNER, INNER)]],
                        rbufs[buf].at[pl.ds(j * INNER, INNER)],
                        sg,
                    )
                )
            for c in copies:
                c.wait()

        def process(c, buf, prefetch, storewait):
            wait_idx(buf)
            if prefetch:
                start_idx(c + 1, 1 - buf)
            if storewait:
                wait_store(buf)
            run_gathers(buf)
            start_store(c, buf)

        start_idx(0, 0)
        process(0, 0, True, False)
        process(1, 1, True, False)

        def body(j, carry):
            process(2 * j, 0, True, True)
            process(2 * j + 1, 1, True, True)
            return carry

        lax.fori_loop(1, N_SUP // 2 - 1, body, 0)

        process(N_SUP - 2, 0, True, True)
        process(N_SUP - 1, 1, False, True)
        wait_store(0)
        wait_store(1)

    return gather_kernel(table, idx_flat)


def _full(shape):
    return pl.BlockSpec(shape, lambda i: (0, 0))


def _rows(width):
    return pl.BlockSpec((P, width), lambda i: (i, 0))


def _pass_a_kernel(g_ref, x_ref, bd1_ref, bd1n_ref, wx_ref, wxn_ref, acc_ref):
    g = g_ref[...]
    x = x_ref[...]
    y1 = jnp.dot(g, bd1_ref[...], preferred_element_type=jnp.float32,
                 precision=lax.Precision.DEFAULT)
    y1 += jnp.dot(x, wx_ref[...], preferred_element_type=jnp.float32,
                  precision=lax.Precision.DEFAULT)
    y1n = jnp.dot(g, bd1n_ref[...], preferred_element_type=jnp.float32,
                  precision=lax.Precision.DEFAULT)
    y1n += jnp.dot(x, wxn_ref[...], preferred_element_type=jnp.float32,
                   precision=lax.Precision.DEFAULT)
    part = jnp.stack(
        [
            jnp.sum(y1, axis=0),
            jnp.sum(y1 * y1, axis=0),
            jnp.sum(y1n, axis=0),
            jnp.sum(y1n * y1n, axis=0),
        ]
    )
    part = jnp.concatenate([part, jnp.zeros((4, LE), jnp.float32)], axis=0)

    @pl.when(pl.program_id(0) == 0)
    def _():
        acc_ref[...] = jnp.zeros_like(acc_ref)

    acc_ref[...] += part


def _pass_b_kernel(g_ref, x_ref, bd1_ref, bd1n_ref, wx_ref, wxn_ref,
                   prm_ref, m2_ref, m2n_ref,
                   eo_ref, s2_ref, s2n_ref, acc_ref):
    g = g_ref[...]
    x = x_ref[...]
    y1 = jnp.dot(g, bd1_ref[...], preferred_element_type=jnp.float32,
                 precision=lax.Precision.DEFAULT)
    y1 += jnp.dot(x, wx_ref[...], preferred_element_type=jnp.float32,
                  precision=lax.Precision.DEFAULT)
    y1n = jnp.dot(g, bd1n_ref[...], preferred_element_type=jnp.float32,
                  precision=lax.Precision.DEFAULT)
    y1n += jnp.dot(x, wxn_ref[...], preferred_element_type=jnp.float32,
                   precision=lax.Precision.DEFAULT)
    xa = y1 * prm_ref[0:1, :] + prm_ref[1:2, :]
    xa = jnp.where(xa > 0, xa, 0.2 * xa)
    eo = y1n * prm_ref[2:3, :] + prm_ref[3:4, :]
    eo = jnp.where(eo > 0, eo, 0.2 * eo)
    eo_ref[...] = eo
    s2 = jnp.dot(xa, m2_ref[...], preferred_element_type=jnp.float32,
                 precision=lax.Precision.DEFAULT)
    s2n = jnp.dot(eo, m2n_ref[...], preferred_element_type=jnp.float32,
                  precision=lax.Precision.DEFAULT)
    s2_ref[...] = s2
    s2n_ref[...] = s2n
    sums = jnp.stack(
        [jnp.sum(s2), jnp.sum(s2 * s2), jnp.sum(s2n), jnp.sum(s2n * s2n)]
    )
    part = jnp.concatenate(
        [jnp.broadcast_to(sums[:, None], (4, 128)),
         jnp.zeros((4, 128), jnp.float32)], axis=0)

    @pl.when(pl.program_id(0) == 0)
    def _():
        acc_ref[...] = jnp.zeros_like(acc_ref)

    acc_ref[...] += part


def _pass_c_kernel(g_ref, x_ref, bd1n_ref, wxn_ref, prm_ref, scal_ref,
                   s2_ref, s2n_ref, exp_ref, msum_ref, out_ref):
    g = g_ref[...]
    x = x_ref[...]
    y1n = jnp.dot(g, bd1n_ref[...], preferred_element_type=jnp.float32,
                  precision=lax.Precision.DEFAULT)
    y1n += jnp.dot(x, wxn_ref[...], preferred_element_type=jnp.float32,
                   precision=lax.Precision.DEFAULT)
    eo = y1n * prm_ref[2:3, :] + prm_ref[3:4, :]
    eo = jnp.where(eo > 0, eo, 0.2 * eo)
    x2 = s2_ref[...] * scal_ref[0:1, 0:1] + scal_ref[0:1, 1:2]
    e2 = s2n_ref[...] * scal_ref[0:1, 2:3] + scal_ref[0:1, 3:4]
    att = x2 + e2
    att = jnp.where(att > 0, att, 0.2 * att)
    att = att - jnp.max(att, axis=1, keepdims=True)
    att = jnp.exp(att)
    att = att / jnp.sum(att, axis=1, keepdims=True)
    att_x = jnp.dot(att, exp_ref[...], preferred_element_type=jnp.float32,
                    precision=lax.Precision.DEFAULT)
    out_ref[...] = jnp.dot(att_x * eo, msum_ref[...],
                           preferred_element_type=jnp.float32,
                           precision=lax.Precision.DEFAULT)


def kernel(x, pos, idx, dis, w1, g1, b1, w2, g2, b2, w1n, g1n, b1n, w2n, g2n, b2n):
    f32 = jnp.float32
    xt = jnp.transpose(x, (0, 2, 1)).reshape(R, C_IN)
    table = jnp.pad(xt, ((0, 0), (0, D - C_IN)))
    idx_flat = (idx + (jnp.arange(B, dtype=idx.dtype) * N)[:, None, None]).reshape(-1)

    w1a_t = jnp.pad(w1[:, :C_IN].T, ((0, D - C_IN), (0, 0)))
    w1na_t = jnp.pad(w1n[:, :C_IN].T, ((0, D - C_IN), (0, 0)))
    d1_t = jnp.pad((w1[:, C_IN:] - w1[:, :C_IN]).T, ((0, D - C_IN), (0, 0)))
    d1n_t = jnp.pad((w1n[:, C_IN:] - w1n[:, :C_IN]).T, ((0, D - C_IN), (0, 0)))
    eye_k = jnp.eye(K, dtype=f32)
    bd1 = jnp.kron(eye_k, w1a_t)
    bd1n = jnp.kron(eye_k, w1na_t)
    wx = jnp.tile(d1_t, (1, K))
    wxn = jnp.tile(d1n_t, (1, K))
    m2 = jnp.kron(eye_k, w2[0][:, None])
    m2n = jnp.kron(eye_k, w2n[0][:, None])
    expand = jnp.kron(eye_k, jnp.ones((1, CH), f32))
    msum = jnp.kron(jnp.ones((K, 1), f32), jnp.eye(CH, dtype=f32))

    g_flat = _sc_gather(table, idx_flat)
    g2d = g_flat.reshape(R, LW)

    acc1 = pl.pallas_call(
        _pass_a_kernel,
        grid=(GRID,),
        in_specs=[_rows(LW), _rows(D), _full((LW, LE)), _full((LW, LE)),
                  _full((D, LE)), _full((D, LE))],
        out_specs=_full((8, LE)),
        out_shape=jax.ShapeDtypeStruct((8, LE), f32),
    )(g2d, table, bd1, bd1n, wx, wxn)

    m = f32(BNK)
    sum1 = acc1[0].reshape(K, CH).sum(0)
    sq1 = acc1[1].reshape(K, CH).sum(0)
    sum1n = acc1[2].reshape(K, CH).sum(0)
    sq1n = acc1[3].reshape(K, CH).sum(0)
    mean1 = sum1 / m
    var1 = sq1 / m - mean1 * mean1
    mean1n = sum1n / m
    var1n = sq1n / m - mean1n * mean1n
    sc1 = g1 / jnp.sqrt(var1 + 1e-5)
    sh1 = b1 - mean1 * sc1
    sc1n = g1n / jnp.sqrt(var1n + 1e-5)
    sh1n = b1n - mean1n * sc1n
    prm = jnp.stack([jnp.tile(sc1, K), jnp.tile(sh1, K),
                     jnp.tile(sc1n, K), jnp.tile(sh1n, K)])

    eo2d, s2, s2n, acc2 = pl.pallas_call(
        _pass_b_kernel,
        grid=(GRID,),
        in_specs=[_rows(LW), _rows(D), _full((LW, LE)), _full((LW, LE)),
                  _full((D, LE)), _full((D, LE)), _full((4, LE)),
                  _full((LE, K)), _full((LE, K))],
        out_specs=[_rows(LE), _rows(K), _rows(K), _full((8, 128))],
        out_shape=[
            jax.ShapeDtypeStruct((R, LE), f32),
            jax.ShapeDtypeStruct((R, K), f32),
            jax.ShapeDtypeStruct((R, K), f32),
            jax.ShapeDtypeStruct((8, 128), f32),
        ],
    )(g2d, table, bd1, bd1n, wx, wxn, prm, m2, m2n)

    mean2 = acc2[0, 0] / m
    var2 = acc2[1, 0] / m - mean2 * mean2
    mean2n = acc2[2, 0] / m
    var2n = acc2[3, 0] / m - mean2n * mean2n
    a2 = g2[0] / jnp.sqrt(var2 + 1e-5)
    c2 = b2[0] - mean2 * a2
    a2n = g2n[0] / jnp.sqrt(var2n + 1e-5)
    c2n = b2n[0] - mean2n * a2n
    scal = jnp.zeros((1, 128), f32)
    scal = scal.at[0, 0].set(a2).at[0, 1].set(c2)
    scal = scal.at[0, 2].set(a2n).at[0, 3].set(c2n)

    out2d = pl.pallas_call(
        _pass_c_kernel,
        grid=(GRID,),
        in_specs=[_rows(LW), _rows(D), _full((LW, LE)), _full((D, LE)),
                  _full((4, LE)), _full((1, 128)), _rows(K), _rows(K),
                  _full((K, LE)), _full((LE, CH))],
        out_specs=_rows(CH),
        out_shape=jax.ShapeDtypeStruct((R, CH), f32),
    )(g2d, table, bd1n, wxn, prm, scal, s2, s2n, expand, msum)

    out = out2d.reshape(B, N, CH)
    edge_o = eo2d.reshape(B, N, K, CH)
    return (out, edge_o)


# speedup vs baseline: 29.7028x; 1.0702x over previous
import functools

import jax
import jax.numpy as jnp
from jax import lax
from jax.experimental import pallas as pl
from jax.experimental.pallas import tpu as pltpu
from jax.experimental.pallas import tpu_sc as plsc

B, C_IN, N, K, CH = 8, 3, 16384, 20, 16
R = B * N
BNK = B * N * K
D = 8
LW = K * D
LE = K * CH
P = 2048
GRID = R // P

NC, NS = 2, 16
NW = NC * NS
E_W = BNK // NW
SUP = 2048
INNER = 128
N_SUP = E_W // SUP


def _sc_gather(table, idx_flat):
    @functools.partial(
        pl.kernel,
        mesh=plsc.VectorSubcoreMesh(core_axis_name="c", subcore_axis_name="s"),
        out_type=jax.ShapeDtypeStruct((BNK, D), jnp.float32),
        scratch_types=[
            pltpu.VMEM((SUP,), jnp.int32),
            pltpu.VMEM((SUP,), jnp.int32),
            pltpu.VMEM((SUP, D), jnp.float32),
            pltpu.VMEM((SUP, D), jnp.float32),
            pltpu.SemaphoreType.DMA,
            pltpu.SemaphoreType.DMA,
            pltpu.SemaphoreType.DMA,
        ],
        compiler_params=pltpu.CompilerParams(use_tc_tiling_on_sc=False),
    )
    def gather_kernel(table_hbm, idx_hbm, out_hbm,
                      ib0, ib1, rb0, rb1, si, sg, ss):
        wid = lax.axis_index("s") * NC + lax.axis_index("c")
        wbase = wid * E_W
        ibufs = (ib0, ib1)
        rbufs = (rb0, rb1)

        def start_idx(c, buf):
            pltpu.async_copy(idx_hbm.at[pl.ds(wbase + c * SUP, SUP)],
                             ibufs[buf], si)

        def wait_idx(buf):
            pltpu.make_async_copy(idx_hbm.at[pl.ds(0, SUP)],
                                  ibufs[buf], si).wait()

        def start_store(c, buf):
            pltpu.async_copy(rbufs[buf],
                             out_hbm.at[pl.ds(wbase + c * SUP, SUP)], ss)

        def wait_store(buf):
            pltpu.make_async_copy(rbufs[buf],
                                  out_hbm.at[pl.ds(0, SUP)], ss).wait()

        def run_gathers(buf):
            copies = []
            for j in range(SUP // INNER):
                copies.append(
                    pltpu.async_copy(
                        table_hbm.at[ibufs[buf].at[pl.ds(j * INNER, INNER)]],
                        rbufs[buf].at[pl.ds(j * INNER, INNER)],
                        sg,
                    )
                )
            for c in copies:
                c.wait()

        def process(c, buf, prefetch, storewait):
            wait_idx(buf)
            if prefetch:
                start_idx(c + 1, 1 - buf)
            if storewait:
                wait_store(buf)
            run_gathers(buf)
            start_store(c, buf)

        start_idx(0, 0)
        process(0, 0, True, False)
        process(1, 1, True, False)

        def body(j, carry):
            process(2 * j, 0, True, True)
            process(2 * j + 1, 1, True, True)
            return carry

        lax.fori_loop(1, N_SUP // 2 - 1, body, 0)

        process(N_SUP - 2, 0, True, True)
        process(N_SUP - 1, 1, False, True)
        wait_store(0)
        wait_store(1)

    return gather_kernel(table, idx_flat)


def _full(shape):
    return pl.BlockSpec(shape, lambda i: (0, 0))


def _rows(width):
    return pl.BlockSpec((P, width), lambda i: (i, 0))


def _pass_a_kernel(g_ref, x_ref, bd1_ref, bd1n_ref, wx_ref, wxn_ref, acc_ref):
    g = g_ref[...]
    x = x_ref[...]
    y1 = jnp.dot(g, bd1_ref[...], preferred_element_type=jnp.float32,
                 precision=lax.Precision.DEFAULT)
    y1 += jnp.dot(x, wx_ref[...], preferred_element_type=jnp.float32,
                  precision=lax.Precision.DEFAULT)
    y1n = jnp.dot(g, bd1n_ref[...], preferred_element_type=jnp.float32,
                  precision=lax.Precision.DEFAULT)
    y1n += jnp.dot(x, wxn_ref[...], preferred_element_type=jnp.float32,
                   precision=lax.Precision.DEFAULT)
    part = jnp.stack(
        [
            jnp.sum(y1, axis=0),
            jnp.sum(y1 * y1, axis=0),
            jnp.sum(y1n, axis=0),
            jnp.sum(y1n * y1n, axis=0),
        ]
    )
    part = jnp.concatenate([part, jnp.zeros((4, LE), jnp.float32)], axis=0)

    @pl.when(pl.program_id(0) == 0)
    def _():
        acc_ref[...] = jnp.zeros_like(acc_ref)

    acc_ref[...] += part


def _pass_b_kernel(g_ref, x_ref, bd1_ref, bd1n_ref, wx_ref, wxn_ref,
                   prm_ref, m2_ref, m2n_ref,
                   eo_ref, s2_ref, s2n_ref, acc_ref):
    g = g_ref[...]
    x = x_ref[...]
    y1 = jnp.dot(g, bd1_ref[...], preferred_element_type=jnp.float32,
                 precision=lax.Precision.DEFAULT)
    y1 += jnp.dot(x, wx_ref[...], preferred_element_type=jnp.float32,
                  precision=lax.Precision.DEFAULT)
    y1n = jnp.dot(g, bd1n_ref[...], preferred_element_type=jnp.float32,
                  precision=lax.Precision.DEFAULT)
    y1n += jnp.dot(x, wxn_ref[...], preferred_element_type=jnp.float32,
                   precision=lax.Precision.DEFAULT)
    xa = y1 * prm_ref[0:1, :] + prm_ref[1:2, :]
    xa = jnp.where(xa > 0, xa, 0.2 * xa)
    eo = y1n * prm_ref[2:3, :] + prm_ref[3:4, :]
    eo = jnp.where(eo > 0, eo, 0.2 * eo)
    eo_ref[...] = eo
    s2 = jnp.dot(xa, m2_ref[...], preferred_element_type=jnp.float32,
                 precision=lax.Precision.DEFAULT)
    s2n = jnp.dot(eo, m2n_ref[...], preferred_element_type=jnp.float32,
                  precision=lax.Precision.DEFAULT)
    s2_ref[...] = s2
    s2n_ref[...] = s2n
    sums = jnp.stack(
        [jnp.sum(s2), jnp.sum(s2 * s2), jnp.sum(s2n), jnp.sum(s2n * s2n)]
    )
    part = jnp.concatenate(
        [jnp.broadcast_to(sums[:, None], (4, 128)),
         jnp.zeros((4, 128), jnp.float32)], axis=0)

    @pl.when(pl.program_id(0) == 0)
    def _():
        acc_ref[...] = jnp.zeros_like(acc_ref)

    acc_ref[...] += part


def _pass_c_kernel(g_ref, x_ref, bd1n_ref, wxn_ref, prm_ref, scal_ref,
                   s2_ref, s2n_ref, exp_ref, msum_ref, out_ref):
    g = g_ref[...]
    x = x_ref[...]
    y1n = jnp.dot(g, bd1n_ref[...], preferred_element_type=jnp.float32,
                  precision=lax.Precision.DEFAULT)
    y1n += jnp.dot(x, wxn_ref[...], preferred_element_type=jnp.float32,
                   precision=lax.Precision.DEFAULT)
    eo = y1n * prm_ref[2:3, :] + prm_ref[3:4, :]
    eo = jnp.where(eo > 0, eo, 0.2 * eo)
    x2 = s2_ref[...] * scal_ref[0:1, 0:1] + scal_ref[0:1, 1:2]
    e2 = s2n_ref[...] * scal_ref[0:1, 2:3] + scal_ref[0:1, 3:4]
    att = x2 + e2
    att = jnp.where(att > 0, att, 0.2 * att)
    att = att - jnp.max(att, axis=1, keepdims=True)
    att = jnp.exp(att)
    att = att / jnp.sum(att, axis=1, keepdims=True)
    att_x = jnp.dot(att, exp_ref[...], preferred_element_type=jnp.float32,
                    precision=lax.Precision.DEFAULT)
    out_ref[...] = jnp.dot(att_x * eo, msum_ref[...],
                           preferred_element_type=jnp.float32,
                           precision=lax.Precision.DEFAULT)


def kernel(x, pos, idx, dis, w1, g1, b1, w2, g2, b2, w1n, g1n, b1n, w2n, g2n, b2n):
    f32 = jnp.float32
    xt = jnp.transpose(x, (0, 2, 1)).reshape(R, C_IN)
    table = jnp.pad(xt, ((0, 0), (0, D - C_IN)))
    idx_flat = (idx + (jnp.arange(B, dtype=idx.dtype) * N)[:, None, None]).reshape(-1)

    w1a_t = jnp.pad(w1[:, :C_IN].T, ((0, D - C_IN), (0, 0)))
    w1na_t = jnp.pad(w1n[:, :C_IN].T, ((0, D - C_IN), (0, 0)))
    d1_t = jnp.pad((w1[:, C_IN:] - w1[:, :C_IN]).T, ((0, D - C_IN), (0, 0)))
    d1n_t = jnp.pad((w1n[:, C_IN:] - w1n[:, :C_IN]).T, ((0, D - C_IN), (0, 0)))
    eye_k = jnp.eye(K, dtype=f32)
    bd1 = jnp.kron(eye_k, w1a_t)
    bd1n = jnp.kron(eye_k, w1na_t)
    wx = jnp.tile(d1_t, (1, K))
    wxn = jnp.tile(d1n_t, (1, K))
    m2 = jnp.kron(eye_k, w2[0][:, None])
    m2n = jnp.kron(eye_k, w2n[0][:, None])
    expand = jnp.kron(eye_k, jnp.ones((1, CH), f32))
    msum = jnp.kron(jnp.ones((K, 1), f32), jnp.eye(CH, dtype=f32))

    g_flat = _sc_gather(table, idx_flat)
    g2d = g_flat.reshape(R, LW)

    acc1 = pl.pallas_call(
        _pass_a_kernel,
        grid=(GRID,),
        in_specs=[_rows(LW), _rows(D), _full((LW, LE)), _full((LW, LE)),
                  _full((D, LE)), _full((D, LE))],
        out_specs=_full((8, LE)),
        out_shape=jax.ShapeDtypeStruct((8, LE), f32),
    )(g2d, table, bd1, bd1n, wx, wxn)

    m = f32(BNK)
    sum1 = acc1[0].reshape(K, CH).sum(0)
    sq1 = acc1[1].reshape(K, CH).sum(0)
    sum1n = acc1[2].reshape(K, CH).sum(0)
    sq1n = acc1[3].reshape(K, CH).sum(0)
    mean1 = sum1 / m
    var1 = sq1 / m - mean1 * mean1
    mean1n = sum1n / m
    var1n = sq1n / m - mean1n * mean1n
    sc1 = g1 / jnp.sqrt(var1 + 1e-5)
    sh1 = b1 - mean1 * sc1
    sc1n = g1n / jnp.sqrt(var1n + 1e-5)
    sh1n = b1n - mean1n * sc1n
    prm = jnp.stack([jnp.tile(sc1, K), jnp.tile(sh1, K),
                     jnp.tile(sc1n, K), jnp.tile(sh1n, K)])

    eo2d, s2, s2n, acc2 = pl.pallas_call(
        _pass_b_kernel,
        grid=(GRID,),
        in_specs=[_rows(LW), _rows(D), _full((LW, LE)), _full((LW, LE)),
                  _full((D, LE)), _full((D, LE)), _full((4, LE)),
                  _full((LE, K)), _full((LE, K))],
        out_specs=[_rows(LE), _rows(K), _rows(K), _full((8, 128))],
        out_shape=[
            jax.ShapeDtypeStruct((R, LE), f32),
            jax.ShapeDtypeStruct((R, K), f32),
            jax.ShapeDtypeStruct((R, K), f32),
            jax.ShapeDtypeStruct((8, 128), f32),
        ],
    )(g2d, table, bd1, bd1n, wx, wxn, prm, m2, m2n)

    mean2 = acc2[0, 0] / m
    var2 = acc2[1, 0] / m - mean2 * mean2
    mean2n = acc2[2, 0] / m
    var2n = acc2[3, 0] / m - mean2n * mean2n
    a2 = g2[0] / jnp.sqrt(var2 + 1e-5)
    c2 = b2[0] - mean2 * a2
    a2n = g2n[0] / jnp.sqrt(var2n + 1e-5)
    c2n = b2n[0] - mean2n * a2n
    scal = jnp.zeros((1, 128), f32)
    scal = scal.at[0, 0].set(a2).at[0, 1].set(c2)
    scal = scal.at[0, 2].set(a2n).at[0, 3].set(c2n)

    out2d = pl.pallas_call(
        _pass_c_kernel,
        grid=(GRID,),
        in_specs=[_rows(LW), _rows(D), _full((LW, LE)), _full((D, LE)),
                  _full((4, LE)), _full((1, 128)), _rows(K), _rows(K),
                  _full((K, LE)), _full((LE, CH))],
        out_specs=_rows(CH),
        out_shape=jax.ShapeDtypeStruct((R, CH), f32),
    )(g2d, table, bd1n, wxn, prm, scal, s2, s2n, expand, msum)

    out = out2d.reshape(B, N, CH)
    edge_o = eo2d.reshape(B, N, K, CH)
    return (out, edge_o)
